# Initial kernel scaffold; baseline (speedup 1.0000x reference)
import jax
import jax.numpy as jnp
from jax.experimental import pallas as pl


def kernel(x, emb, w_ih, w_hh, b_ih, b_hh):
    raise NotImplementedError("write your pallas kernel here")


# speedup vs baseline: 3.1471x; 3.1471x over previous
import functools

import jax
import jax.numpy as jnp
from jax import lax
from jax.experimental import pallas as pl
from jax.experimental.pallas import tpu as pltpu
from jax.experimental.pallas import tpu_sc as plsc

VOCAB = 100000
ES = 32
HS = 64
B = 4096
T = 50

_NC = 2
_NS = 16
_NW = _NC * _NS
_ROWS = B * T
_RPW = _ROWS // _NW
_GRP = 128
_NGRP = _RPW // _GRP
_FIRE = 10
_NOUT = _NGRP // _FIRE


def _sc_gather_body(emb_hbm, idx_hbm, out_hbm, idx_v, rows_v, sem):
  wid = lax.axis_index("s") * _NC + lax.axis_index("c")
  pltpu.sync_copy(idx_hbm.at[wid], idx_v)

  def outer(o, carry):
    copies = []
    for j in range(_FIRE):
      cp = pltpu.async_copy(
          emb_hbm.at[idx_v.at[o * _FIRE + j]], rows_v.at[j], sem)
      copies.append(cp)
    for cp in copies:
      cp.wait()
    pltpu.sync_copy(rows_v, out_hbm.at[pl.ds(wid * _NGRP + o * _FIRE, _FIRE)])
    return carry

  lax.fori_loop(0, _NOUT, outer, 0)


@functools.cache
def _sc_gather():
  return functools.partial(
      pl.kernel,
      out_type=jax.ShapeDtypeStruct((_ROWS // _GRP, _GRP, ES), jnp.float32),
      mesh=plsc.VectorSubcoreMesh(core_axis_name="c", subcore_axis_name="s"),
      scratch_types=[
          pltpu.VMEM((_NGRP, _GRP), jnp.int32),
          pltpu.VMEM((_FIRE, _GRP, ES), jnp.float32),
          pltpu.SemaphoreType.DMA,
      ],
      compiler_params=pltpu.CompilerParams(use_tc_tiling_on_sc=False),
  )(_sc_gather_body)


_BB = 512


def _gru_body(e_ref, x_ref, wiT_ref, whT_ref, bih_ref, bhh_ref, out_ref):
  xb = x_ref[...]
  l = jnp.sum((xb != 0).astype(jnp.int32), axis=1, keepdims=True)
  wiT = wiT_ref[...]
  whT = whT_ref[...]
  bih = bih_ref[...]
  bhh = bhh_ref[...]

  def step(t, h):
    e_t = e_ref[t]
    gi = jnp.dot(e_t, wiT, preferred_element_type=jnp.float32) + bih
    gh = jnp.dot(h, whT, preferred_element_type=jnp.float32) + bhh
    i_r = gi[:, :HS]
    i_z = gi[:, HS:2 * HS]
    i_n = gi[:, 2 * HS:]
    h_r = gh[:, :HS]
    h_z = gh[:, HS:2 * HS]
    h_n = gh[:, 2 * HS:]
    r = jax.nn.sigmoid(i_r + h_r)
    z = jax.nn.sigmoid(i_z + h_z)
    n = jnp.tanh(i_n + r * h_n)
    h_new = (1.0 - z) * n + z * h
    return jnp.where(t < l, h_new, h)

  h = lax.fori_loop(0, T, step, jnp.zeros((_BB, HS), jnp.float32))
  out_ref[...] = h


def _gru(e_tm, x, wiT, whT, bih, bhh, interpret=False):
  grid = (B // _BB,)
  return pl.pallas_call(
      _gru_body,
      grid=grid,
      in_specs=[
          pl.BlockSpec((T, _BB, ES), lambda i: (0, i, 0)),
          pl.BlockSpec((_BB, T), lambda i: (i, 0)),
          pl.BlockSpec((ES, 3 * HS), lambda i: (0, 0)),
          pl.BlockSpec((HS, 3 * HS), lambda i: (0, 0)),
          pl.BlockSpec((1, 3 * HS), lambda i: (0, 0)),
          pl.BlockSpec((1, 3 * HS), lambda i: (0, 0)),
      ],
      out_specs=pl.BlockSpec((_BB, HS), lambda i: (i, 0)),
      out_shape=jax.ShapeDtypeStruct((B, HS), jnp.float32),
      compiler_params=pltpu.CompilerParams(
          dimension_semantics=("arbitrary",),
      ),
      interpret=interpret,
  )(e_tm, x, wiT, whT, bih, bhh)


def kernel(x, emb, w_ih, w_hh, b_ih, b_hh):
  idx3 = x.T.reshape(_NW, _NGRP, _GRP)
  e3 = _sc_gather()(emb, idx3)
  e_tm = e3.reshape(T, B, ES)
  return _gru(e_tm, x, w_ih.T, w_hh.T, b_ih[None, :], b_hh[None, :])


# speedup vs baseline: 4.7568x; 1.5115x over previous
import functools

import jax
import jax.numpy as jnp
from jax import lax
from jax.experimental import pallas as pl
from jax.experimental.pallas import tpu as pltpu
from jax.experimental.pallas import tpu_sc as plsc

VOCAB = 100000
ES = 32
HS = 64
B = 4096
T = 50

_NC = 2
_NS = 16
_NW = _NC * _NS
_ROWS = B * T
_RPW = _ROWS // _NW
_GRP = 128
_NGRP = _RPW // _GRP
_FIRE = 10
_NOUT = _NGRP // _FIRE


def _sc_gather_body(emb_hbm, idx_hbm, out_hbm, idx_v, rows_v, sem):
  wid = lax.axis_index("s") * _NC + lax.axis_index("c")
  pltpu.sync_copy(idx_hbm.at[wid], idx_v)

  def outer(o, carry):
    copies = []
    for j in range(_FIRE):
      cp = pltpu.async_copy(
          emb_hbm.at[idx_v.at[o * _FIRE + j]], rows_v.at[j], sem)
      copies.append(cp)
    for cp in copies:
      cp.wait()
    pltpu.sync_copy(rows_v, out_hbm.at[pl.ds(wid * _NGRP + o * _FIRE, _FIRE)])
    return carry

  lax.fori_loop(0, _NOUT, outer, 0)


@functools.cache
def _sc_gather():
  return functools.partial(
      pl.kernel,
      out_type=jax.ShapeDtypeStruct((_ROWS // _GRP, _GRP, ES), jnp.float32),
      mesh=plsc.VectorSubcoreMesh(core_axis_name="c", subcore_axis_name="s"),
      scratch_types=[
          pltpu.VMEM((_NGRP, _GRP), jnp.int32),
          pltpu.VMEM((_FIRE, _GRP, ES), jnp.float32),
          pltpu.SemaphoreType.DMA,
      ],
      compiler_params=pltpu.CompilerParams(use_tc_tiling_on_sc=False),
  )(_sc_gather_body)


_BB = 512


_F = 4
_FH = _F * HS
_FE = _F * ES


def _gru_body(e_ref, xf_ref, wi_ref, wh_ref, gib_ref, bhn_ref, m_ref,
              out_ref, gi_s):
  bq = _BB // _F
  ecnt = (xf_ref[...] != 0).astype(jnp.float32)
  lfold = jnp.dot(ecnt, m_ref[...],
                  preferred_element_type=jnp.float32).astype(jnp.int32)
  e2d = e_ref[...].reshape(T * bq, _FE)
  gi_s[...] = (jnp.dot(e2d, wi_ref[...], preferred_element_type=jnp.float32)
               + gib_ref[...]).reshape(T, bq, 3 * _FH)
  wh = wh_ref[...]
  bhn = bhn_ref[...]

  def step(t, h):
    gi = gi_s[t]
    gh = jnp.dot(h, wh, preferred_element_type=jnp.float32)
    r = jax.nn.sigmoid(gi[:, :_FH] + gh[:, :_FH])
    z = jax.nn.sigmoid(gi[:, _FH:2 * _FH] + gh[:, _FH:2 * _FH])
    n = jnp.tanh(gi[:, 2 * _FH:] + r * (gh[:, 2 * _FH:] + bhn))
    h_new = (1.0 - z) * n + z * h
    return jnp.where(t < lfold, h_new, h)

  h = lax.fori_loop(0, T, step, jnp.zeros((bq, _FH), jnp.float32))
  out_ref[...] = h


def _gru(e4, xf, wi4, wh4, gib4, bhn4, mmat, interpret=False):
  grid = (B // _BB,)
  bq = _BB // _F
  return pl.pallas_call(
      _gru_body,
      grid=grid,
      in_specs=[
          pl.BlockSpec((T, bq, _FE), lambda i: (0, i, 0)),
          pl.BlockSpec((bq, _F * T), lambda i: (i, 0)),
          pl.BlockSpec((_FE, 3 * _FH), lambda i: (0, 0)),
          pl.BlockSpec((_FH, 3 * _FH), lambda i: (0, 0)),
          pl.BlockSpec((1, 3 * _FH), lambda i: (0, 0)),
          pl.BlockSpec((1, _FH), lambda i: (0, 0)),
          pl.BlockSpec((_F * T, _FH), lambda i: (0, 0)),
      ],
      out_specs=pl.BlockSpec((bq, _FH), lambda i: (i, 0)),
      out_shape=jax.ShapeDtypeStruct((B // _F, _FH), jnp.float32),
      scratch_shapes=[pltpu.VMEM((T, bq, 3 * _FH), jnp.float32)],
      compiler_params=pltpu.CompilerParams(
          dimension_semantics=("arbitrary",),
      ),
      interpret=interpret,
  )(e4, xf, wi4, wh4, gib4, bhn4, mmat)


def kernel(x, emb, w_ih, w_hh, b_ih, b_hh):
  idx3 = x.T.reshape(_NW, _NGRP, _GRP)
  e3 = _sc_gather()(emb, idx3)
  e4 = e3.reshape(T, B // _F, _FE)
  xf = x.reshape(B // _F, _F * T)

  eye = jnp.eye(_F, dtype=jnp.float32)
  wi4 = jnp.concatenate(
      [jnp.kron(eye, w_ih[g * HS:(g + 1) * HS, :].T) for g in range(3)],
      axis=1)
  wh4 = jnp.concatenate(
      [jnp.kron(eye, w_hh[g * HS:(g + 1) * HS, :].T) for g in range(3)],
      axis=1)
  gib4 = jnp.concatenate([
      jnp.tile(b_ih[0:HS] + b_hh[0:HS], _F),
      jnp.tile(b_ih[HS:2 * HS] + b_hh[HS:2 * HS], _F),
      jnp.tile(b_ih[2 * HS:], _F),
  ])[None, :]
  bhn4 = jnp.tile(b_hh[2 * HS:], _F)[None, :]
  kk = jnp.arange(_F * T) // T
  jj = jnp.arange(_FH) // HS
  mmat = (kk[:, None] == jj[None, :]).astype(jnp.float32)

  h4 = _gru(e4, xf, wi4, wh4, gib4, bhn4, mmat)
  return h4.reshape(B, HS)


# speedup vs baseline: 6.1305x; 1.2888x over previous
import functools

import jax
import jax.numpy as jnp
from jax import lax
from jax.experimental import pallas as pl
from jax.experimental.pallas import tpu as pltpu
from jax.experimental.pallas import tpu_sc as plsc

VOCAB = 100000
ES = 32
HS = 64
B = 4096
T = 50

_NC = 2
_NS = 16
_NW = _NC * _NS
_ROWS = B * T
_RPW = _ROWS // _NW
_GRP = 128
_NGRP = _RPW // _GRP
_FIRE = 10
_NOUT = _NGRP // _FIRE


def _sc_gather_body(emb_hbm, idx_hbm, out_hbm, idx_v, rows_v, sem):
  wid = lax.axis_index("s") * _NC + lax.axis_index("c")
  pltpu.sync_copy(idx_hbm.at[wid], idx_v)

  def outer(o, carry):
    copies = []
    for j in range(_FIRE):
      cp = pltpu.async_copy(
          emb_hbm.at[idx_v.at[o * _FIRE + j]], rows_v.at[j], sem)
      copies.append(cp)
    for cp in copies:
      cp.wait()
    pltpu.sync_copy(rows_v, out_hbm.at[pl.ds(wid * _NGRP + o * _FIRE, _FIRE)])
    return carry

  lax.fori_loop(0, _NOUT, outer, 0)


@functools.cache
def _sc_gather():
  return functools.partial(
      pl.kernel,
      out_type=jax.ShapeDtypeStruct((_ROWS // _GRP, _GRP, ES), jnp.float32),
      mesh=plsc.VectorSubcoreMesh(core_axis_name="c", subcore_axis_name="s"),
      scratch_types=[
          pltpu.VMEM((_NGRP, _GRP), jnp.int32),
          pltpu.VMEM((_FIRE, _GRP, ES), jnp.float32),
          pltpu.SemaphoreType.DMA,
      ],
      compiler_params=pltpu.CompilerParams(use_tc_tiling_on_sc=False),
  )(_sc_gather_body)


_BB = 512


_F = 4
_FH = _F * HS
_FE = _F * ES
_BQ = B // _F


def _gru_body(e_ref, xf_ref, wrz_ref, win_ref, whn_ref, brz_ref, bin_ref,
              bhn_ref, m_ref, out_ref):
  ecnt = (xf_ref[...] != 0).astype(jnp.float32)
  lfold = jnp.dot(ecnt, m_ref[...],
                  preferred_element_type=jnp.float32).astype(jnp.int32)
  wrz = wrz_ref[...]
  win = win_ref[...]
  whn = whn_ref[...]
  brz = brz_ref[...]
  bin_ = bin_ref[...]
  bhn = bhn_ref[...]

  def step(t, h):
    e_t = e_ref[t]
    eh = jnp.concatenate([e_t, h], axis=1)
    rz = jnp.dot(eh, wrz, preferred_element_type=jnp.float32) + brz
    r = jax.nn.sigmoid(rz[:, :_FH])
    z = jax.nn.sigmoid(rz[:, _FH:])
    gin = jnp.dot(e_t, win, preferred_element_type=jnp.float32) + bin_
    ghn = jnp.dot(h, whn, preferred_element_type=jnp.float32) + bhn
    n = jnp.tanh(gin + r * ghn)
    h_new = (1.0 - z) * n + z * h
    return jnp.where(t < lfold, h_new, h)

  h = lax.fori_loop(0, T, step, jnp.zeros((_BQ, _FH), jnp.float32))
  out_ref[...] = h


def _gru(e4, xf, wrz, win, whn, brz, bin_, bhn, mmat, interpret=False):
  return pl.pallas_call(
      _gru_body,
      grid=(1,),
      in_specs=[
          pl.BlockSpec((T, _BQ, _FE), lambda i: (0, 0, 0)),
          pl.BlockSpec((_BQ, _F * T), lambda i: (0, 0)),
          pl.BlockSpec((_FE + _FH, 2 * _FH), lambda i: (0, 0)),
          pl.BlockSpec((_FE, _FH), lambda i: (0, 0)),
          pl.BlockSpec((_FH, _FH), lambda i: (0, 0)),
          pl.BlockSpec((1, 2 * _FH), lambda i: (0, 0)),
          pl.BlockSpec((1, _FH), lambda i: (0, 0)),
          pl.BlockSpec((1, _FH), lambda i: (0, 0)),
          pl.BlockSpec((_F * T, _FH), lambda i: (0, 0)),
      ],
      out_specs=pl.BlockSpec((_BQ, _FH), lambda i: (0, 0)),
      out_shape=jax.ShapeDtypeStruct((_BQ, _FH), jnp.float32),
      compiler_params=pltpu.CompilerParams(
          dimension_semantics=("arbitrary",),
      ),
      interpret=interpret,
  )(e4, xf, wrz, win, whn, brz, bin_, bhn, mmat)


def kernel(x, emb, w_ih, w_hh, b_ih, b_hh):
  idx3 = x.T.reshape(_NW, _NGRP, _GRP)
  e3 = _sc_gather()(emb, idx3)
  e4 = e3.reshape(T, B // _F, _FE)
  xf = x.reshape(B // _F, _F * T)

  eye = jnp.eye(_F, dtype=jnp.float32)
  kr = lambda w: jnp.kron(eye, w)
  wir, wiz, win_ = (w_ih[g * HS:(g + 1) * HS, :].T for g in range(3))
  whr, whz, whn_ = (w_hh[g * HS:(g + 1) * HS, :].T for g in range(3))
  wrz = jnp.concatenate([
      jnp.concatenate([kr(wir), kr(wiz)], axis=1),
      jnp.concatenate([kr(whr), kr(whz)], axis=1),
  ], axis=0)
  win4 = kr(win_)
  whn4 = kr(whn_)
  brz = jnp.concatenate([
      jnp.tile(b_ih[0:HS] + b_hh[0:HS], _F),
      jnp.tile(b_ih[HS:2 * HS] + b_hh[HS:2 * HS], _F),
  ])[None, :]
  bin4 = jnp.tile(b_ih[2 * HS:], _F)[None, :]
  bhn4 = jnp.tile(b_hh[2 * HS:], _F)[None, :]
  kk = jnp.arange(_F * T) // T
  jj = jnp.arange(_FH) // HS
  mmat = (kk[:, None] == jj[None, :]).astype(jnp.float32)

  h4 = _gru(e4, xf, wrz, win4, whn4, brz, bin4, bhn4, mmat)
  return h4.reshape(B, HS)
